# baseline (speedup 1.0000x reference)
import functools

import jax
import jax.numpy as jnp
from jax import lax
from jax.experimental import pallas as pl
from jax.experimental.pallas import tpu as pltpu
from jax.experimental.pallas import tpu_sc as plsc

_B = 128
_HW = 1024
_DIM = 32
_K = 512
_N = _B * _HW
_HB = _B // 2

_TILES = 2
_RB = _HB // _TILES


def _argmax_body(xt_ref, ct_ref, lab_ref):
    ct = ct_ref[...].astype(jnp.bfloat16)
    for j in range(_RB):
        x = xt_ref[j]
        nrm = jnp.maximum(
            jnp.sqrt(jnp.sum(x * x, axis=0, keepdims=True)), 1e-12)
        xn = x / nrm
        sim = lax.dot_general(ct, xn.astype(jnp.bfloat16),
                              (((0,), (0,)), ((), ())),
                              preferred_element_type=jnp.float32)
        lab = jnp.argmax(sim, axis=0)
        lab_ref[j] = lab.astype(jnp.int32)


def _make_argmax_call(half):
    off = half * _TILES
    return pl.pallas_call(
        _argmax_body,
        grid=(_TILES,),
        in_specs=[
            pl.BlockSpec((_RB, _DIM, _HW), lambda i: (i + off, 0, 0)),
            pl.BlockSpec((_DIM, _K), lambda i: (0, 0)),
        ],
        out_specs=pl.BlockSpec((_RB, _HW), lambda i: (i, 0)),
        out_shape=jax.ShapeDtypeStruct((_HB, _HW), jnp.int32),
    )


_argmax_halves = (_make_argmax_call(0), _make_argmax_call(1))

_NC = 2
_NS = 16
_NW = _NC * _NS
_ROWS_W = _HB // _NW
_CHUNK = _ROWS_W * _HW
_L = 16


def _sc_gather_body(labels_hbm, table_hbm, out_hbm, table_v, idx_v, out_v):
    wid = lax.axis_index("s") * _NC + lax.axis_index("c")
    row0 = wid * _ROWS_W
    pltpu.sync_copy(table_hbm, table_v)
    for j in range(_ROWS_W):
        pltpu.sync_copy(labels_hbm.at[row0 + j],
                        idx_v.at[pl.ds(j * _HW, _HW)])

    @plsc.parallel_loop(0, _CHUNK, step=_L, unroll=8)
    def body(i):
        sl = pl.ds(i, _L)
        out_v[sl] = plsc.load_gather(table_v, [idx_v[sl]])

    for j in range(_ROWS_W):
        pltpu.sync_copy(out_v.at[pl.ds(j * _HW, _HW)],
                        out_hbm.at[row0 + j])


@functools.cache
def _sc_gather():
    return pl.kernel(
        _sc_gather_body,
        out_type=jax.ShapeDtypeStruct((_HB, _HW), jnp.int32),
        mesh=plsc.VectorSubcoreMesh(core_axis_name="c", subcore_axis_name="s",
                                    num_cores=_NC, num_subcores=_NS),
        compiler_params=pltpu.CompilerParams(needs_layout_passes=False),
        scratch_types=[
            pltpu.VMEM((_K,), jnp.int32),
            pltpu.VMEM((_CHUNK,), jnp.int32),
            pltpu.VMEM((_CHUNK,), jnp.int32),
        ],
    )


def kernel(features, cluster_centers, pseudo_assignment):
    feats_t = jnp.swapaxes(features, 1, 2)
    centers_t = cluster_centers.T
    pa = pseudo_assignment.astype(jnp.int32)
    gather = _sc_gather()
    lab0 = _argmax_halves[0](feats_t, centers_t)
    seg0 = gather(lab0, pa)
    lab1 = _argmax_halves[1](feats_t, centers_t)
    seg1 = gather(lab1, pa)
    pseudo_segs_pred = jnp.concatenate([lab0, lab1], axis=0)
    segs_pred = jnp.concatenate([seg0, seg1], axis=0)
    return pseudo_segs_pred, segs_pred


# speedup vs baseline: 1.0753x; 1.0753x over previous
import functools

import jax
import jax.numpy as jnp
from jax import lax
from jax.experimental import pallas as pl
from jax.experimental.pallas import tpu as pltpu
from jax.experimental.pallas import tpu_sc as plsc

_B = 128
_HW = 1024
_DIM = 32
_K = 512
_N = _B * _HW

_TILES = 4
_R = _N // _TILES
_RB = _R // _HW


def _argmax_body(xt_ref, ct_ref, lab_ref):
    ct = ct_ref[...].astype(jnp.bfloat16)
    for j in range(_RB):
        x = xt_ref[j]
        nrm = jnp.maximum(
            jnp.sqrt(jnp.sum(x * x, axis=0, keepdims=True)), 1e-12)
        xn = x / nrm
        sim = lax.dot_general(ct, xn.astype(jnp.bfloat16),
                              (((0,), (0,)), ((), ())),
                              preferred_element_type=jnp.float32)
        lab = jnp.argmax(sim, axis=0)
        lab_ref[j] = lab.astype(jnp.int32)


_argmax_call = pl.pallas_call(
    _argmax_body,
    grid=(_TILES,),
    in_specs=[
        pl.BlockSpec((_RB, _DIM, _HW), lambda i: (i, 0, 0)),
        pl.BlockSpec((_DIM, _K), lambda i: (0, 0)),
    ],
    out_specs=pl.BlockSpec((_RB, _HW), lambda i: (i, 0)),
    out_shape=jax.ShapeDtypeStruct((_B, _HW), jnp.int32),
)

_NC = 2
_NS = 16
_NW = _NC * _NS
_ROWS_W = _B // _NW
_CHUNK = _ROWS_W * _HW
_L = 16


def _sc_gather_body(labels_hbm, table_hbm, out_hbm, table_v, idx_v, out_v):
    wid = lax.axis_index("s") * _NC + lax.axis_index("c")
    row0 = wid * _ROWS_W
    pltpu.sync_copy(table_hbm, table_v)
    for j in range(_ROWS_W):
        pltpu.sync_copy(labels_hbm.at[row0 + j],
                        idx_v.at[pl.ds(j * _HW, _HW)])

    @plsc.parallel_loop(0, _CHUNK, step=_L, unroll=8)
    def body(i):
        sl = pl.ds(i, _L)
        out_v[sl] = plsc.load_gather(table_v, [idx_v[sl]])
    for j in range(_ROWS_W):
        pltpu.sync_copy(out_v.at[pl.ds(j * _HW, _HW)],
                        out_hbm.at[row0 + j])


@functools.cache
def _sc_gather():
    return pl.kernel(
        _sc_gather_body,
        out_type=jax.ShapeDtypeStruct((_B, _HW), jnp.int32),
        mesh=plsc.VectorSubcoreMesh(core_axis_name="c", subcore_axis_name="s",
                                    num_cores=_NC, num_subcores=_NS),
        compiler_params=pltpu.CompilerParams(needs_layout_passes=False),
        scratch_types=[
            pltpu.VMEM((_K,), jnp.int32),
            pltpu.VMEM((_CHUNK,), jnp.int32),
            pltpu.VMEM((_CHUNK,), jnp.int32),
        ],
    )


def kernel(features, cluster_centers, pseudo_assignment):
    feats_t = jnp.swapaxes(features, 1, 2)
    centers_t = cluster_centers.T
    pseudo_segs_pred = _argmax_call(feats_t, centers_t)
    segs_pred = _sc_gather()(pseudo_segs_pred,
                             pseudo_assignment.astype(jnp.int32))
    return pseudo_segs_pred, segs_pred
